# Initial kernel scaffold; baseline (speedup 1.0000x reference)
import jax
import jax.numpy as jnp
from jax.experimental import pallas as pl


def kernel(node_feature, edge_indices, W_line, b_line, W_self, b_self):
    raise NotImplementedError("write your pallas kernel here")


# speedup vs baseline: 5.1204x; 5.1204x over previous
import functools

import jax
import jax.numpy as jnp
from jax import lax
from jax.experimental import pallas as pl
from jax.experimental.pallas import tpu as pltpu
from jax.experimental.pallas import tpu_sc as plsc

NC = 2
NS = 16
NW = NC * NS
CHUNK = 128


def _round_up(x, m):
    return (x + m - 1) // m * m


def _sc_accumulate(node_feature, src_p, dst_p, zeros_nd, zeros_1, n_pad, ew):
    n, d = node_feature.shape
    rps = n_pad // NS
    mesh = plsc.VectorSubcoreMesh(core_axis_name="c", subcore_axis_name="s")

    @functools.partial(
        pl.kernel,
        mesh=mesh,
        out_type=[
            jax.ShapeDtypeStruct((NC * n_pad, d), jnp.float32),
            jax.ShapeDtypeStruct((NC * n_pad,), jnp.float32),
        ],
        scratch_types=[
            pltpu.VMEM((CHUNK,), jnp.int32),
            pltpu.VMEM((CHUNK,), jnp.int32),
            pltpu.VMEM((CHUNK, d), jnp.float32),
            pltpu.VMEM((CHUNK,), jnp.float32),
            pltpu.VMEM_SHARED((n_pad, d), jnp.float32),
            pltpu.VMEM_SHARED((n_pad,), jnp.float32),
            pltpu.SemaphoreType.DMA,
        ],
    )
    def accum(x_hbm, src_hbm, dst_hbm, znd_hbm, z1_hbm, psum_hbm, pcnt_hbm,
              idx_s, idx_d, rows, ones_v, acc, cnt, sem):
        c = lax.axis_index("c")
        s = lax.axis_index("s")
        wid = c * NS + s
        for j in range(CHUNK // 16):
            ones_v[pl.ds(16 * j, 16)] = jnp.ones((16,), jnp.float32)
        pltpu.sync_copy(znd_hbm.at[pl.ds(s * rps, rps)], acc.at[pl.ds(s * rps, rps)])
        pltpu.sync_copy(z1_hbm.at[pl.ds(s * rps, rps)], cnt.at[pl.ds(s * rps, rps)])
        plsc.subcore_barrier()

        base = wid * ew

        def chunk_body(j, carry):
            b = base + j * CHUNK
            pltpu.sync_copy(src_hbm.at[pl.ds(b, CHUNK)], idx_s)
            pltpu.sync_copy(dst_hbm.at[pl.ds(b, CHUNK)], idx_d)
            pltpu.async_copy(x_hbm.at[idx_s], rows, sem).wait()
            pltpu.sync_copy(rows, acc.at[idx_d], add=True)
            pltpu.sync_copy(ones_v, cnt.at[idx_d], add=True)
            return carry

        lax.fori_loop(0, ew // CHUNK, chunk_body, 0)
        plsc.subcore_barrier()
        pltpu.sync_copy(acc.at[pl.ds(s * rps, rps)],
                        psum_hbm.at[pl.ds(c * n_pad + s * rps, rps)])
        pltpu.sync_copy(cnt.at[pl.ds(s * rps, rps)],
                        pcnt_hbm.at[pl.ds(c * n_pad + s * rps, rps)])

    return accum(node_feature, src_p, dst_p, zeros_nd, zeros_1)


def _tc_combine_matmul(psum, pcnt2, x, w_line, w_self, b_line, b_self, n_pad):
    n, d = x.shape
    br = 2048
    grid = -(-n // br)
    nb = n_pad // br

    def body(p0_ref, p1_ref, cnt_ref, x_ref, wl_ref, ws_ref, bl_ref, bs_ref,
             out_ref):
        cnt = cnt_ref[0, :] + cnt_ref[1, :]
        rec = 1.0 / jnp.maximum(cnt, 1.0)
        mean = (p0_ref[...] + p1_ref[...]) * rec[:, None]
        dn = (((1,), (1,)), ((), ()))
        acc = lax.dot_general(mean, wl_ref[...], dn,
                              preferred_element_type=jnp.float32)
        acc = acc + lax.dot_general(x_ref[...], ws_ref[...], dn,
                                    preferred_element_type=jnp.float32)
        out_ref[...] = acc + bl_ref[...] + bs_ref[...]

    return pl.pallas_call(
        body,
        grid=(grid,),
        in_specs=[
            pl.BlockSpec((br, d), lambda i: (i, 0)),
            pl.BlockSpec((br, d), lambda i: (i + nb, 0)),
            pl.BlockSpec((2, br), lambda i: (0, i)),
            pl.BlockSpec((br, d), lambda i: (i, 0)),
            pl.BlockSpec((d, d), lambda i: (0, 0)),
            pl.BlockSpec((d, d), lambda i: (0, 0)),
            pl.BlockSpec((1, d), lambda i: (0, 0)),
            pl.BlockSpec((1, d), lambda i: (0, 0)),
        ],
        out_specs=pl.BlockSpec((br, d), lambda i: (i, 0)),
        out_shape=jax.ShapeDtypeStruct((n, d), jnp.float32),
    )(psum, psum, pcnt2, x, w_line, w_self, b_line, b_self)


def kernel(node_feature, edge_indices, W_line, b_line, W_self, b_self):
    n, d = node_feature.shape
    e = edge_indices.shape[1]
    ew = _round_up(-(-e // NW), CHUNK)
    e_pad = NW * ew
    n_pad = _round_up(n + 1, 2 * NW * 16)

    pad = e_pad - e
    src_p = jnp.concatenate([edge_indices[0], jnp.zeros((pad,), jnp.int32)])
    dst_p = jnp.concatenate([edge_indices[1], jnp.full((pad,), n, jnp.int32)])
    zeros_nd = jnp.zeros((n_pad, d), jnp.float32)
    zeros_1 = jnp.zeros((n_pad,), jnp.float32)

    psum, pcnt = _sc_accumulate(node_feature, src_p, dst_p, zeros_nd, zeros_1,
                                n_pad, ew)
    return _tc_combine_matmul(psum, pcnt.reshape(NC, n_pad), node_feature,
                              W_line, W_self, b_line.reshape(1, d),
                              b_self.reshape(1, d), n_pad)
